# baseline (speedup 1.0000x reference)
import jax
import jax.numpy as jnp
from jax.experimental import pallas as pl
from jax.experimental.pallas import tpu as pltpu

N = 4096
B = 8
ABLK = 128
NBLK = N // ABLK
CONV_STEPS = 5 * NBLK
KBLK = 256
NK = N // KBLK
C2 = 16
HEAD_STEPS = NK * C2


def _mega_body(a_ref, xv_ref, ws1_ref, ws2_ref, b1_ref, b2_ref, wf1_ref,
               bf1_ref, wf2_ref, bf2_ref, wf3_ref, bf3_ref, out_ref,
               A, U12, P0, Y1, Y2, V12, S1, S2, Q0, G2, GX, acc):
    j = pl.program_id(0)
    ph = jnp.minimum(j // NBLK, 5)
    r = pl.multiple_of((j % NBLK) * ABLK, ABLK)

    @pl.when(ph == 0)
    def _():
        A[pl.ds(r, ABLK), :] = a_ref[...].astype(jnp.bfloat16)
        t = jnp.dot(xv_ref[pl.ds(r, ABLK), :], ws1_ref[...],
                    preferred_element_type=jnp.float32)
        U12[pl.ds(r, ABLK), :] = t[:, :512].astype(jnp.bfloat16)
        P0[pl.ds(r, ABLK), :] = t[:, 512:].astype(jnp.bfloat16)

    @pl.when(ph == 1)
    def _():
        y = jnp.dot(A[pl.ds(r, ABLK), :], U12[...],
                    preferred_element_type=jnp.float32)
        Y1[pl.ds(r, ABLK), :] = y[:, :256].astype(jnp.bfloat16)
        Y2[pl.ds(r, ABLK), :] = y[:, 256:].astype(jnp.bfloat16)

    @pl.when(ph == 2)
    def _():
        z2 = jnp.dot(A[pl.ds(r, ABLK), :], Y2[...],
                     preferred_element_type=jnp.float32)
        h = jnp.maximum(P0[pl.ds(r, ABLK), :].astype(jnp.float32)
                        + Y1[pl.ds(r, ABLK), :].astype(jnp.float32)
                        + 2.0 * z2 + b1_ref[...], 0.0)
        t2 = jnp.dot(h.astype(jnp.bfloat16), ws2_ref[...],
                     preferred_element_type=jnp.float32)
        V12[pl.ds(r, ABLK), :] = t2[:, :256].astype(jnp.bfloat16)
        Q0[pl.ds(r, ABLK), :] = t2[:, 256:].astype(jnp.bfloat16)

    @pl.when(ph == 3)
    def _():
        s = jnp.dot(A[pl.ds(r, ABLK), :], V12[...],
                    preferred_element_type=jnp.float32)
        S1[pl.ds(r, ABLK), :] = s[:, :128].astype(jnp.bfloat16)
        S2[pl.ds(r, ABLK), :] = s[:, 128:].astype(jnp.bfloat16)

    @pl.when(ph == 4)
    def _():
        t2 = jnp.dot(A[pl.ds(r, ABLK), :], S2[...],
                     preferred_element_type=jnp.float32)
        G2[pl.ds(r, ABLK), :] = jnp.maximum(
            Q0[pl.ds(r, ABLK), :].astype(jnp.float32)
            + S1[pl.ds(r, ABLK), :].astype(jnp.float32)
            + 2.0 * t2 + b2_ref[...], 0.0)

    @pl.when(ph == 5)
    def _():
        jj = j - CONV_STEPS
        k = jj // C2
        c = jj % C2
        rk = pl.multiple_of(k * KBLK, KBLK)

        @pl.when(jj == 0)
        def _():
            acc[...] = jnp.zeros_like(acc)

        @pl.when(c == 0)
        def _():
            GX[...] = jnp.transpose(G2[pl.ds(rk, KBLK), :])

        lhs = GX[pl.ds(pl.multiple_of(c * B, B), B), :]
        acc[...] += jnp.dot(lhs.astype(jnp.bfloat16),
                            wf1_ref[:, 0, 0, :].astype(jnp.bfloat16),
                            preferred_element_type=jnp.float32)

        @pl.when(jj == HEAD_STEPS - 1)
        def _():
            h1 = jnp.maximum(acc[...] + bf1_ref[...], 0.0)
            h2 = jnp.maximum(
                jnp.dot(h1, wf2_ref[...], preferred_element_type=jnp.float32)
                + bf2_ref[...], 0.0)
            lg = jnp.dot(h2, wf3_ref[...],
                         preferred_element_type=jnp.float32) + bf3_ref[...]
            m = jnp.max(lg, axis=-1, keepdims=True)
            e = jnp.exp(lg - m)
            out_ref[...] = e / jnp.sum(e, axis=-1, keepdims=True)


def _full(shape):
    nd = len(shape)
    return pl.BlockSpec(shape, lambda *_, _nd=nd: (0,) * _nd)


def _kron8(w):
    return jnp.kron(jnp.eye(B, dtype=w.dtype), w)


def _cmajor(w):
    fin, fout = w.shape
    t = jnp.eye(B, dtype=w.dtype)[:, None, None, :] * w[None, :, :, None]
    return t.reshape(B * fin, fout * B)


def kernel(x, a, W1, b1, W2, b2, Wf1, bf1, Wf2, bf2, Wf3, bf3):
    C1 = W1.shape[2]
    M1, M2, M3 = Wf1.shape[1], Wf2.shape[1], Wf3.shape[1]

    xv = x.transpose(1, 0, 2).reshape(N, B * 64).astype(jnp.bfloat16)
    ws1 = jnp.concatenate(
        [_kron8(W1[1]), _kron8(W1[2]), _kron8(W1[0] - W1[2])],
        axis=1).astype(jnp.bfloat16)
    ws2 = jnp.concatenate(
        [_cmajor(W2[1]), _cmajor(W2[2]), _cmajor(W2[0] - W2[2])],
        axis=1).astype(jnp.bfloat16)
    b1t = jnp.tile(b1, B).reshape(1, B * C1)
    b2r = jnp.repeat(b2, B).reshape(1, C2 * B)
    wf1v = Wf1.reshape(N, C2, 1, M1)

    bf16 = jnp.bfloat16
    f32 = jnp.float32

    def _a_idx(j):
        return (jnp.minimum(j, NBLK - 1), 0)

    def _wf1_idx(j):
        jj = jnp.clip(j - CONV_STEPS, 0, HEAD_STEPS - 1)
        return (jj // C2, jj % C2, 0, 0)

    out = pl.pallas_call(
        _mega_body,
        grid=(CONV_STEPS + HEAD_STEPS,),
        in_specs=[
            pl.BlockSpec((ABLK, N), _a_idx),
            _full((N, B * 64)), _full((512, 768)), _full((256, 384)),
            _full((1, B * C1)), _full((1, C2 * B)),
            pl.BlockSpec((KBLK, 1, 1, M1), _wf1_idx),
            _full((1, M1)), _full((M1, M2)), _full((1, M2)),
            _full((M2, M3)), _full((1, M3)),
        ],
        out_specs=pl.BlockSpec((B, M3), lambda j: (0, 0)),
        out_shape=jax.ShapeDtypeStruct((B, M3), f32),
        scratch_shapes=[
            pltpu.VMEM((N, N), bf16),
            pltpu.VMEM((N, 512), bf16),
            pltpu.VMEM((N, 256), bf16),
            pltpu.VMEM((N, 256), bf16),
            pltpu.VMEM((N, 256), bf16),
            pltpu.VMEM((N, 256), bf16),
            pltpu.VMEM((N, 128), bf16),
            pltpu.VMEM((N, 128), bf16),
            pltpu.VMEM((N, 128), bf16),
            pltpu.VMEM((N, C2 * B), f32),
            pltpu.VMEM((C2 * B, KBLK), f32),
            pltpu.VMEM((B, M1), f32),
        ],
        compiler_params=pltpu.CompilerParams(
            dimension_semantics=("arbitrary",),
            vmem_limit_bytes=64 * 1024 * 1024),
    )(a, xv, ws1, ws2, b1t, b2r, wf1v, bf1.reshape(1, M1), Wf2,
      bf2.reshape(1, M2), Wf3, bf3.reshape(1, M3))
    return out


# speedup vs baseline: 4.8614x; 4.8614x over previous
import jax
import jax.numpy as jnp
from jax.experimental import pallas as pl
from jax.experimental.pallas import tpu as pltpu

N = 4096
B = 8
ABLK = 256
NBLK = N // ABLK


def _mega_body(a_ref, xv_ref, ws1_ref, ws2_ref, b1_ref, b2_ref, g_ref,
               A, U12, P0, Y1, Y2, V12, S1, S2, Q0):
    j = pl.program_id(0)
    ph = j // NBLK
    r = pl.multiple_of((j % NBLK) * ABLK, ABLK)

    @pl.when(ph == 0)
    def _():
        A[pl.ds(r, ABLK), :] = a_ref[...].astype(jnp.bfloat16)
        t = jnp.dot(xv_ref[pl.ds(r, ABLK), :], ws1_ref[...],
                    preferred_element_type=jnp.float32)
        U12[pl.ds(r, ABLK), :] = t[:, :512].astype(jnp.bfloat16)
        P0[pl.ds(r, ABLK), :] = t[:, 512:].astype(jnp.bfloat16)

    @pl.when(ph == 1)
    def _():
        y = jnp.dot(A[pl.ds(r, ABLK), :], U12[...],
                    preferred_element_type=jnp.float32)
        Y1[pl.ds(r, ABLK), :] = y[:, :256].astype(jnp.bfloat16)
        Y2[pl.ds(r, ABLK), :] = y[:, 256:].astype(jnp.bfloat16)

    @pl.when(ph == 2)
    def _():
        z2 = jnp.dot(A[pl.ds(r, ABLK), :], Y2[...],
                     preferred_element_type=jnp.float32)
        h = jnp.maximum(P0[pl.ds(r, ABLK), :].astype(jnp.float32)
                        + Y1[pl.ds(r, ABLK), :].astype(jnp.float32)
                        + 2.0 * z2 + b1_ref[...], 0.0)
        t2 = jnp.dot(h.astype(jnp.bfloat16), ws2_ref[...],
                     preferred_element_type=jnp.float32)
        V12[pl.ds(r, ABLK), :] = t2[:, :256].astype(jnp.bfloat16)
        Q0[pl.ds(r, ABLK), :] = t2[:, 256:].astype(jnp.bfloat16)

    @pl.when(ph == 3)
    def _():
        s = jnp.dot(A[pl.ds(r, ABLK), :], V12[...],
                    preferred_element_type=jnp.float32)
        S1[pl.ds(r, ABLK), :] = s[:, :128].astype(jnp.bfloat16)
        S2[pl.ds(r, ABLK), :] = s[:, 128:].astype(jnp.bfloat16)

    @pl.when(ph == 4)
    def _():
        t2 = jnp.dot(A[pl.ds(r, ABLK), :], S2[...],
                     preferred_element_type=jnp.float32)
        g_ref[...] = jnp.maximum(
            Q0[pl.ds(r, ABLK), :].astype(jnp.float32)
            + S1[pl.ds(r, ABLK), :].astype(jnp.float32)
            + 2.0 * t2 + b2_ref[...], 0.0)


def _head_body(gt_ref, wf1_ref, bf1_ref, wf2_ref, bf2_ref, wf3_ref, bf3_ref,
               out_ref, acc_ref):
    k = pl.program_id(0)
    nk = pl.num_programs(0)

    @pl.when(k == 0)
    def _():
        acc_ref[...] = jnp.zeros_like(acc_ref)

    acc_ref[...] += jax.lax.dot_general(
        gt_ref[...], wf1_ref[...], (((0,), (0,)), ((), ())),
        preferred_element_type=jnp.float32)

    @pl.when(k == nk - 1)
    def _():
        h1 = jnp.maximum(acc_ref[...] + bf1_ref[...], 0.0)
        h2 = jnp.maximum(
            jnp.dot(h1, wf2_ref[...], preferred_element_type=jnp.float32)
            + bf2_ref[...], 0.0)
        lg = jnp.dot(h2, wf3_ref[...],
                     preferred_element_type=jnp.float32) + bf3_ref[...]
        m = jnp.max(lg, axis=-1, keepdims=True)
        e = jnp.exp(lg - m)
        out_ref[...] = e / jnp.sum(e, axis=-1, keepdims=True)


def _full(shape):
    nd = len(shape)
    return pl.BlockSpec(shape, lambda *_, _nd=nd: (0,) * _nd)


def _kron8(w):
    return jnp.kron(jnp.eye(B, dtype=w.dtype), w)


def _cmajor(w):
    fin, fout = w.shape
    t = jnp.eye(B, dtype=w.dtype)[:, None, None, :] * w[None, :, :, None]
    return t.reshape(B * fin, fout * B)


def kernel(x, a, W1, b1, W2, b2, Wf1, bf1, Wf2, bf2, Wf3, bf3):
    C1 = W1.shape[2]
    C2 = W2.shape[2]
    M1, M2, M3 = Wf1.shape[1], Wf2.shape[1], Wf3.shape[1]

    xv = x.transpose(1, 0, 2).reshape(N, B * 64).astype(jnp.bfloat16)
    ws1 = jnp.concatenate(
        [_kron8(W1[1]), _kron8(W1[2]), _kron8(W1[0] - W1[2])],
        axis=1).astype(jnp.bfloat16)
    ws2 = jnp.concatenate(
        [_cmajor(W2[1]), _cmajor(W2[2]), _cmajor(W2[0] - W2[2])],
        axis=1).astype(jnp.bfloat16)
    b1t = jnp.tile(b1, B).reshape(1, B * C1)
    b2r = jnp.repeat(b2, B).reshape(1, C2 * B)

    bf16 = jnp.bfloat16
    f32 = jnp.float32
    g = pl.pallas_call(
        _mega_body,
        grid=(5 * NBLK,),
        in_specs=[
            pl.BlockSpec((ABLK, N), lambda j: (jnp.minimum(j, NBLK - 1), 0)),
            _full((N, B * 64)), _full((512, 768)), _full((256, 384)),
            _full((1, B * C1)), _full((1, C2 * B)),
        ],
        out_specs=pl.BlockSpec(
            (ABLK, C2 * B), lambda j: (jnp.clip(j - 4 * NBLK, 0, NBLK - 1), 0)),
        out_shape=jax.ShapeDtypeStruct((N, C2 * B), f32),
        scratch_shapes=[
            pltpu.VMEM((N, N), bf16),
            pltpu.VMEM((N, 512), bf16),
            pltpu.VMEM((N, 256), bf16),
            pltpu.VMEM((N, 256), bf16),
            pltpu.VMEM((N, 256), bf16),
            pltpu.VMEM((N, 256), bf16),
            pltpu.VMEM((N, 128), bf16),
            pltpu.VMEM((N, 128), bf16),
            pltpu.VMEM((N, 128), bf16),
        ],
        compiler_params=pltpu.CompilerParams(
            dimension_semantics=("arbitrary",),
            vmem_limit_bytes=64 * 1024 * 1024),
    )(a, xv, ws1, ws2, b1t, b2r)

    gt = g.reshape(N * C2, B)
    kblk = 4096
    out = pl.pallas_call(
        _head_body,
        grid=((N * C2) // kblk,),
        in_specs=[pl.BlockSpec((kblk, B), lambda k: (k, 0)),
                  pl.BlockSpec((kblk, M1), lambda k: (k, 0)),
                  _full((1, M1)), _full((M1, M2)), _full((1, M2)),
                  _full((M2, M3)), _full((1, M3))],
        out_specs=pl.BlockSpec((B, M3), lambda k: (0, 0)),
        out_shape=jax.ShapeDtypeStruct((B, M3), f32),
        scratch_shapes=[pltpu.VMEM((B, M1), f32)],
        compiler_params=pltpu.CompilerParams(
            dimension_semantics=("arbitrary",),
            vmem_limit_bytes=64 * 1024 * 1024),
    )(gt, Wf1, bf1.reshape(1, M1), Wf2, bf2.reshape(1, M2),
      Wf3, bf3.reshape(1, M3))
    return out


# speedup vs baseline: 5.0252x; 1.0337x over previous
import jax
import jax.numpy as jnp
from jax.experimental import pallas as pl
from jax.experimental.pallas import tpu as pltpu

N = 4096
B = 8
ABLK = 256
NBLK = N // ABLK


def _mega_body(a_ref, xv_ref, ws1_ref, ws2_ref, b1_ref, b2_ref, g_ref,
               A, U12, P0, Y1, Y2, V12, S1, S2, Q0):
    j = pl.program_id(0)
    ph = j // NBLK
    r = pl.multiple_of((j % NBLK) * ABLK, ABLK)

    @pl.when(ph == 0)
    def _():
        @pl.when(j == 0)
        def _():
            def _proj(kk, carry):
                rr = pl.multiple_of(kk * 128, 128)
                t = jnp.dot(xv_ref[pl.ds(rr, 128), :], ws1_ref[...],
                            preferred_element_type=jnp.float32)
                U12[pl.ds(rr, 128), :] = t[:, :512].astype(jnp.bfloat16)
                P0[pl.ds(rr, 128), :] = t[:, 512:].astype(jnp.bfloat16)
                return carry
            jax.lax.fori_loop(0, N // 128, _proj, 0)

        ab = a_ref[...].astype(jnp.bfloat16)
        A[pl.ds(r, ABLK), :] = ab
        y = jnp.dot(ab, U12[...],
                    preferred_element_type=jnp.float32)
        Y1[pl.ds(r, ABLK), :] = y[:, :256].astype(jnp.bfloat16)
        Y2[pl.ds(r, ABLK), :] = y[:, 256:].astype(jnp.bfloat16)

    @pl.when(ph == 1)
    def _():
        z2 = jnp.dot(A[pl.ds(r, ABLK), :], Y2[...],
                     preferred_element_type=jnp.float32)
        h = jnp.maximum(P0[pl.ds(r, ABLK), :].astype(jnp.float32)
                        + Y1[pl.ds(r, ABLK), :].astype(jnp.float32)
                        + 2.0 * z2 + b1_ref[...], 0.0)
        t2 = jnp.dot(h.astype(jnp.bfloat16), ws2_ref[...],
                     preferred_element_type=jnp.float32)
        V12[pl.ds(r, ABLK), :] = t2[:, :256].astype(jnp.bfloat16)
        Q0[pl.ds(r, ABLK), :] = t2[:, 256:].astype(jnp.bfloat16)

    @pl.when(ph == 2)
    def _():
        s = jnp.dot(A[pl.ds(r, ABLK), :], V12[...],
                    preferred_element_type=jnp.float32)
        S1[pl.ds(r, ABLK), :] = s[:, :128].astype(jnp.bfloat16)
        S2[pl.ds(r, ABLK), :] = s[:, 128:].astype(jnp.bfloat16)

    @pl.when(ph == 3)
    def _():
        t2 = jnp.dot(A[pl.ds(r, ABLK), :], S2[...],
                     preferred_element_type=jnp.float32)
        g_ref[...] = jnp.maximum(
            Q0[pl.ds(r, ABLK), :].astype(jnp.float32)
            + S1[pl.ds(r, ABLK), :].astype(jnp.float32)
            + 2.0 * t2 + b2_ref[...], 0.0)


def _head_body(gt_ref, wf1_ref, bf1_ref, wf2_ref, bf2_ref, wf3_ref, bf3_ref,
               out_ref, acc_ref):
    k = pl.program_id(0)
    nk = pl.num_programs(0)

    @pl.when(k == 0)
    def _():
        acc_ref[...] = jnp.zeros_like(acc_ref)

    acc_ref[...] += jax.lax.dot_general(
        gt_ref[...], wf1_ref[...], (((0,), (0,)), ((), ())),
        preferred_element_type=jnp.float32)

    @pl.when(k == nk - 1)
    def _():
        h1 = jnp.maximum(acc_ref[...] + bf1_ref[...], 0.0)
        h2 = jnp.maximum(
            jnp.dot(h1, wf2_ref[...], preferred_element_type=jnp.float32)
            + bf2_ref[...], 0.0)
        lg = jnp.dot(h2, wf3_ref[...],
                     preferred_element_type=jnp.float32) + bf3_ref[...]
        m = jnp.max(lg, axis=-1, keepdims=True)
        e = jnp.exp(lg - m)
        out_ref[...] = e / jnp.sum(e, axis=-1, keepdims=True)


def _full(shape):
    nd = len(shape)
    return pl.BlockSpec(shape, lambda *_, _nd=nd: (0,) * _nd)


def _kron8(w):
    return jnp.kron(jnp.eye(B, dtype=w.dtype), w)


def _cmajor(w):
    fin, fout = w.shape
    t = jnp.eye(B, dtype=w.dtype)[:, None, None, :] * w[None, :, :, None]
    return t.reshape(B * fin, fout * B)


def kernel(x, a, W1, b1, W2, b2, Wf1, bf1, Wf2, bf2, Wf3, bf3):
    C1 = W1.shape[2]
    C2 = W2.shape[2]
    M1, M2, M3 = Wf1.shape[1], Wf2.shape[1], Wf3.shape[1]

    xv = x.transpose(1, 0, 2).reshape(N, B * 64).astype(jnp.bfloat16)
    ws1 = jnp.concatenate(
        [_kron8(W1[1]), _kron8(W1[2]), _kron8(W1[0] - W1[2])],
        axis=1).astype(jnp.bfloat16)
    ws2 = jnp.concatenate(
        [_cmajor(W2[1]), _cmajor(W2[2]), _cmajor(W2[0] - W2[2])],
        axis=1).astype(jnp.bfloat16)
    b1t = jnp.tile(b1, B).reshape(1, B * C1)
    b2r = jnp.repeat(b2, B).reshape(1, C2 * B)

    bf16 = jnp.bfloat16
    f32 = jnp.float32
    g = pl.pallas_call(
        _mega_body,
        grid=(4 * NBLK,),
        in_specs=[
            pl.BlockSpec((ABLK, N), lambda j: (jnp.minimum(j, NBLK - 1), 0)),
            _full((N, B * 64)), _full((512, 768)), _full((256, 384)),
            _full((1, B * C1)), _full((1, C2 * B)),
        ],
        out_specs=pl.BlockSpec(
            (ABLK, C2 * B), lambda j: (jnp.clip(j - 3 * NBLK, 0, NBLK - 1), 0)),
        out_shape=jax.ShapeDtypeStruct((N, C2 * B), f32),
        scratch_shapes=[
            pltpu.VMEM((N, N), bf16),
            pltpu.VMEM((N, 512), bf16),
            pltpu.VMEM((N, 256), bf16),
            pltpu.VMEM((N, 256), bf16),
            pltpu.VMEM((N, 256), bf16),
            pltpu.VMEM((N, 256), bf16),
            pltpu.VMEM((N, 128), bf16),
            pltpu.VMEM((N, 128), bf16),
            pltpu.VMEM((N, 128), bf16),
        ],
        compiler_params=pltpu.CompilerParams(
            dimension_semantics=("arbitrary",),
            vmem_limit_bytes=64 * 1024 * 1024),
    )(a, xv, ws1, ws2, b1t, b2r)

    gt = g.reshape(N * C2, B)
    kblk = 8192
    out = pl.pallas_call(
        _head_body,
        grid=((N * C2) // kblk,),
        in_specs=[pl.BlockSpec((kblk, B), lambda k: (k, 0)),
                  pl.BlockSpec((kblk, M1), lambda k: (k, 0)),
                  _full((1, M1)), _full((M1, M2)), _full((1, M2)),
                  _full((M2, M3)), _full((1, M3))],
        out_specs=pl.BlockSpec((B, M3), lambda k: (0, 0)),
        out_shape=jax.ShapeDtypeStruct((B, M3), f32),
        scratch_shapes=[pltpu.VMEM((B, M1), f32)],
        compiler_params=pltpu.CompilerParams(
            dimension_semantics=("arbitrary",),
            vmem_limit_bytes=64 * 1024 * 1024),
    )(gt, Wf1, bf1.reshape(1, M1), Wf2, bf2.reshape(1, M2),
      Wf3, bf3.reshape(1, M3))
    return out


# speedup vs baseline: 5.3871x; 1.0720x over previous
import jax
import jax.numpy as jnp
from jax.experimental import pallas as pl
from jax.experimental.pallas import tpu as pltpu

N = 4096
B = 8
ABLK = 256
NBLK = N // ABLK


def _mega_body(a_ref, xv_ref, ws1_ref, ws2_ref, b1_ref, b2_ref, g_ref,
               A, U12, P0, Y1, Y2, V12, S1, S2, Q0):
    j = pl.program_id(0)
    ph = jnp.where(j < NBLK, 0, 1 + (j - NBLK) // 4)
    r = pl.multiple_of((j % NBLK) * ABLK, ABLK)
    rp = pl.multiple_of(((j - NBLK) % 4) * 1024, 1024)

    @pl.when(ph == 0)
    def _():
        @pl.when(j == 0)
        def _():
            def _proj(kk, carry):
                rr = pl.multiple_of(kk * 128, 128)
                t = jnp.dot(xv_ref[pl.ds(rr, 128), :], ws1_ref[...],
                            preferred_element_type=jnp.float32)
                U12[pl.ds(rr, 128), :] = t[:, :512].astype(jnp.bfloat16)
                P0[pl.ds(rr, 128), :] = t[:, 512:].astype(jnp.bfloat16)
                return carry
            jax.lax.fori_loop(0, N // 128, _proj, 0)

        ab = a_ref[...].astype(jnp.bfloat16)
        A[pl.ds(r, ABLK), :] = ab
        y = jnp.dot(ab, U12[...],
                    preferred_element_type=jnp.float32)
        Y1[pl.ds(r, ABLK), :] = y[:, :256].astype(jnp.bfloat16)
        Y2[pl.ds(r, ABLK), :] = y[:, 256:].astype(jnp.bfloat16)

    @pl.when(ph == 1)
    def _():
        z2 = jnp.dot(A[pl.ds(rp, 1024), :], Y2[...],
                     preferred_element_type=jnp.float32)
        h = jnp.maximum(P0[pl.ds(rp, 1024), :].astype(jnp.float32)
                        + Y1[pl.ds(rp, 1024), :].astype(jnp.float32)
                        + 2.0 * z2 + b1_ref[...], 0.0)
        t2 = jnp.dot(h.astype(jnp.bfloat16), ws2_ref[...],
                     preferred_element_type=jnp.float32)
        V12[pl.ds(rp, 1024), :] = t2[:, :256].astype(jnp.bfloat16)
        Q0[pl.ds(rp, 1024), :] = t2[:, 256:].astype(jnp.bfloat16)

    @pl.when(ph == 2)
    def _():
        s = jnp.dot(A[pl.ds(rp, 1024), :], V12[...],
                    preferred_element_type=jnp.float32)
        S1[pl.ds(rp, 1024), :] = s[:, :128].astype(jnp.bfloat16)
        S2[pl.ds(rp, 1024), :] = s[:, 128:].astype(jnp.bfloat16)

    @pl.when(ph == 3)
    def _():
        t2 = jnp.dot(A[pl.ds(rp, 1024), :], S2[...],
                     preferred_element_type=jnp.float32)
        g_ref[...] = jnp.maximum(
            Q0[pl.ds(rp, 1024), :].astype(jnp.float32)
            + S1[pl.ds(rp, 1024), :].astype(jnp.float32)
            + 2.0 * t2 + b2_ref[...], 0.0)


def _head_body(gt_ref, wf1_ref, bf1_ref, wf2_ref, bf2_ref, wf3_ref, bf3_ref,
               out_ref, acc_ref):
    k = pl.program_id(0)
    nk = pl.num_programs(0)

    @pl.when(k == 0)
    def _():
        acc_ref[...] = jnp.zeros_like(acc_ref)

    acc_ref[...] += jax.lax.dot_general(
        gt_ref[...], wf1_ref[...], (((0,), (0,)), ((), ())),
        preferred_element_type=jnp.float32)

    @pl.when(k == nk - 1)
    def _():
        h1 = jnp.maximum(acc_ref[...] + bf1_ref[...], 0.0)
        h2 = jnp.maximum(
            jnp.dot(h1, wf2_ref[...], preferred_element_type=jnp.float32)
            + bf2_ref[...], 0.0)
        lg = jnp.dot(h2, wf3_ref[...],
                     preferred_element_type=jnp.float32) + bf3_ref[...]
        m = jnp.max(lg, axis=-1, keepdims=True)
        e = jnp.exp(lg - m)
        out_ref[...] = e / jnp.sum(e, axis=-1, keepdims=True)


def _full(shape):
    nd = len(shape)
    return pl.BlockSpec(shape, lambda *_, _nd=nd: (0,) * _nd)


def _kron8(w):
    return jnp.kron(jnp.eye(B, dtype=w.dtype), w)


def _cmajor(w):
    fin, fout = w.shape
    t = jnp.eye(B, dtype=w.dtype)[:, None, None, :] * w[None, :, :, None]
    return t.reshape(B * fin, fout * B)


def kernel(x, a, W1, b1, W2, b2, Wf1, bf1, Wf2, bf2, Wf3, bf3):
    C1 = W1.shape[2]
    C2 = W2.shape[2]
    M1, M2, M3 = Wf1.shape[1], Wf2.shape[1], Wf3.shape[1]

    xv = x.transpose(1, 0, 2).reshape(N, B * 64).astype(jnp.bfloat16)
    ws1 = jnp.concatenate(
        [_kron8(W1[1]), _kron8(W1[2]), _kron8(W1[0] - W1[2])],
        axis=1).astype(jnp.bfloat16)
    ws2 = jnp.concatenate(
        [_cmajor(W2[1]), _cmajor(W2[2]), _cmajor(W2[0] - W2[2])],
        axis=1).astype(jnp.bfloat16)
    b1t = jnp.tile(b1, B).reshape(1, B * C1)
    b2r = jnp.repeat(b2, B).reshape(1, C2 * B)

    bf16 = jnp.bfloat16
    f32 = jnp.float32
    g = pl.pallas_call(
        _mega_body,
        grid=(NBLK + 12,),
        in_specs=[
            pl.BlockSpec((ABLK, N), lambda j: (jnp.minimum(j, NBLK - 1), 0)),
            _full((N, B * 64)), _full((512, 768)), _full((256, 384)),
            _full((1, B * C1)), _full((1, C2 * B)),
        ],
        out_specs=pl.BlockSpec(
            (1024, C2 * B), lambda j: (jnp.clip(j - (NBLK + 8), 0, 3), 0)),
        out_shape=jax.ShapeDtypeStruct((N, C2 * B), f32),
        scratch_shapes=[
            pltpu.VMEM((N, N), bf16),
            pltpu.VMEM((N, 512), bf16),
            pltpu.VMEM((N, 256), bf16),
            pltpu.VMEM((N, 256), bf16),
            pltpu.VMEM((N, 256), bf16),
            pltpu.VMEM((N, 256), bf16),
            pltpu.VMEM((N, 128), bf16),
            pltpu.VMEM((N, 128), bf16),
            pltpu.VMEM((N, 128), bf16),
        ],
        compiler_params=pltpu.CompilerParams(
            dimension_semantics=("arbitrary",),
            vmem_limit_bytes=64 * 1024 * 1024),
    )(a, xv, ws1, ws2, b1t, b2r)

    gt = g.reshape(N * C2, B)
    kblk = 8192
    out = pl.pallas_call(
        _head_body,
        grid=((N * C2) // kblk,),
        in_specs=[pl.BlockSpec((kblk, B), lambda k: (k, 0)),
                  pl.BlockSpec((kblk, M1), lambda k: (k, 0)),
                  _full((1, M1)), _full((M1, M2)), _full((1, M2)),
                  _full((M2, M3)), _full((1, M3))],
        out_specs=pl.BlockSpec((B, M3), lambda k: (0, 0)),
        out_shape=jax.ShapeDtypeStruct((B, M3), f32),
        scratch_shapes=[pltpu.VMEM((B, M1), f32)],
        compiler_params=pltpu.CompilerParams(
            dimension_semantics=("arbitrary",),
            vmem_limit_bytes=64 * 1024 * 1024),
    )(gt, Wf1, bf1.reshape(1, M1), Wf2, bf2.reshape(1, M2),
      Wf3, bf3.reshape(1, M3))
    return out
